# baseline (speedup 1.0000x reference)
import jax
import jax.numpy as jnp
from jax.experimental import pallas as pl
from jax.experimental.pallas import tpu as pltpu

EPS = 1e-8
NTAGS = 8
SQRT_TINV = 3.798282186859221

BM = 2048
BN = 1024
BNORM = 1024


def _normalize_kernel(x_ref, out_ref):
    x = x_ref[...]
    norm = jnp.sqrt(jnp.sum(x * x, axis=1, keepdims=True))
    scale = SQRT_TINV / jnp.maximum(norm, EPS)
    out_ref[...] = (x * scale).astype(jnp.bfloat16)


def _info_nce_body(nj, ni, xi_ref, xj_ref, ct_ref, rt_ref, out_ref,
                   sim_a, sim_b, r_acc, diag_acc, loss_acc):
    s = pl.program_id(0)
    tag_iota_c = jax.lax.broadcasted_iota(jnp.int32, (BN, NTAGS), 1)
    tag_iota_r = jax.lax.broadcasted_iota(jnp.int32, (BM, NTAGS), 1)

    p2 = jnp.maximum(s - 1, 0)
    ip = p2 // nj
    jp = jax.lax.rem(p2, nj)

    def step(sim_dst, sim_src):
        sim_dst[...] = jax.lax.dot_general(
            xi_ref[...], xj_ref[...], (((1,), (1,)), ((), ())),
            preferred_element_type=jnp.float32).astype(jnp.bfloat16)

        e_bf = jnp.exp2(sim_src[...]).astype(jnp.bfloat16)
        onehot = (ct_ref[0, :][:, None] == tag_iota_c).astype(jnp.bfloat16)
        r = jax.lax.dot_general(
            e_bf, onehot, (((1,), (0,)), ((), ())),
            preferred_element_type=jnp.float32)
        r_acc[...] = jnp.where(jp == 0, r, r_acc[...] + r)

        @pl.when(jp == 0)
        def _reset_diag():
            diag_acc[...] = jnp.zeros_like(diag_acc)

        @pl.when((jp * BN < ip * BM + BM) & (ip * BM < jp * BN + BN))
        def _diag():
            row_g = jax.lax.broadcasted_iota(jnp.int32, (BM, BN), 0) + ip * BM
            col_g = jax.lax.broadcasted_iota(jnp.int32, (BM, BN), 1) + jp * BN
            d = jnp.sum(
                jnp.where(row_g == col_g, e_bf.astype(jnp.float32), 0.0),
                axis=1, keepdims=True)
            diag_acc[...] += d

        @pl.when((jp == nj - 1) & (s > 0))
        def _finalize_rows():
            rfull = r_acc[...]
            de = diag_acc[...]
            sel = rt_ref[0, :][:, None] == tag_iota_r
            den = jnp.sum(rfull, axis=1, keepdims=True) - de
            num = jnp.sum(jnp.where(sel, rfull, 0.0),
                          axis=1, keepdims=True) - de
            valid = num > 0.0
            num_safe = jnp.where(valid, num, 1.0)
            den_safe = jnp.where(den > 0.0, den, 1.0)
            losses = -jnp.log(num_safe / den_safe)
            loss_sum = jnp.sum(jnp.where(valid, losses, 0.0))
            cnt = jnp.sum(valid.astype(jnp.float32))

            @pl.when(ip == 0)
            def _():
                loss_acc[0, 0] = loss_sum
                loss_acc[0, 1] = cnt

            @pl.when(ip != 0)
            def _():
                loss_acc[0, 0] += loss_sum
                loss_acc[0, 1] += cnt

            @pl.when(ip == ni - 1)
            def _():
                out_ref[0, 0] = loss_acc[0, 0] / jnp.maximum(
                    loss_acc[0, 1], 1.0)

    @pl.when(jax.lax.rem(s, 2) == 0)
    def _even():
        step(sim_a, sim_b)

    @pl.when(jax.lax.rem(s, 2) == 1)
    def _odd():
        step(sim_b, sim_a)


def kernel(representations, ne_tags):
    n, d = representations.shape
    tags = ne_tags.astype(jnp.int32).reshape(1, n)

    xn = pl.pallas_call(
        _normalize_kernel,
        grid=(n // BNORM,),
        in_specs=[pl.BlockSpec((BNORM, d), lambda i: (i, 0))],
        out_specs=pl.BlockSpec((BNORM, d), lambda i: (i, 0)),
        out_shape=jax.ShapeDtypeStruct((n, d), jnp.bfloat16),
    )(representations)

    ni = n // BM
    nj = n // BN

    def body(*refs):
        _info_nce_body(nj, ni, *refs)

    rem = jax.lax.rem
    out = pl.pallas_call(
        body,
        grid=(ni * nj + 1,),
        in_specs=[
            pl.BlockSpec((BM, d), lambda s: (jnp.minimum(s // nj, ni - 1), 0)),
            pl.BlockSpec((BN, d), lambda s: (rem(s, nj), 0)),
            pl.BlockSpec((1, BN),
                         lambda s: (0, rem(jnp.maximum(s - 1, 0), nj))),
            pl.BlockSpec((1, BM),
                         lambda s: (0, jnp.maximum(s - 1, 0) // nj)),
        ],
        out_specs=pl.BlockSpec(
            (1, 2), lambda s: (0, 0), memory_space=pltpu.SMEM),
        out_shape=jax.ShapeDtypeStruct((1, 2), jnp.float32),
        scratch_shapes=[
            pltpu.VMEM((BM, BN), jnp.bfloat16),
            pltpu.VMEM((BM, BN), jnp.bfloat16),
            pltpu.VMEM((BM, NTAGS), jnp.float32),
            pltpu.VMEM((BM, 1), jnp.float32),
            pltpu.SMEM((1, 2), jnp.float32),
        ],
        compiler_params=pltpu.CompilerParams(
            dimension_semantics=("arbitrary",)),
    )(xn, xn, tags, tags)
    return out[0, 0]


# speedup vs baseline: 1.1904x; 1.1904x over previous
import jax
import jax.numpy as jnp
from jax.experimental import pallas as pl
from jax.experimental.pallas import tpu as pltpu

EPS = 1e-8
NTAGS = 8
SQRT_TINV = 3.798282186859221

BM = 1024
BNORM = 1024


def _normalize_kernel(x_ref, out_ref):
    x = x_ref[...]
    norm = jnp.sqrt(jnp.sum(x * x, axis=1, keepdims=True))
    scale = SQRT_TINV / jnp.maximum(norm, EPS)
    out_ref[...] = (x * scale).astype(jnp.bfloat16)


def _row_block_kernel(xi_ref, xn_ref, tags_ref, out_ref):
    i = pl.program_id(0)
    n = xn_ref.shape[0]

    sim = jax.lax.dot_general(
        xi_ref[...], xn_ref[...], (((1,), (1,)), ((), ())),
        preferred_element_type=jnp.float32)
    e_bf = jnp.exp2(sim).astype(jnp.bfloat16)

    ct = tags_ref[0, :]
    onehot = (ct[:, None] ==
              jax.lax.broadcasted_iota(jnp.int32, (n, NTAGS), 1)
              ).astype(jnp.bfloat16)
    r = jax.lax.dot_general(
        e_bf, onehot, (((1,), (0,)), ((), ())),
        preferred_element_type=jnp.float32)

    row_g = jax.lax.broadcasted_iota(jnp.int32, (BM, n), 0) + i * BM
    col_g = jax.lax.broadcasted_iota(jnp.int32, (BM, n), 1)
    de = jnp.sum(jnp.where(row_g == col_g, e_bf.astype(jnp.float32), 0.0),
                 axis=1, keepdims=True)

    rt = tags_ref[0, pl.ds(i * BM, BM)]
    sel = (rt[:, None] ==
           jax.lax.broadcasted_iota(jnp.int32, (BM, NTAGS), 1))
    den = jnp.sum(r, axis=1, keepdims=True) - de
    num = jnp.sum(jnp.where(sel, r, 0.0), axis=1, keepdims=True) - de
    valid = num > 0.0
    num_safe = jnp.where(valid, num, 1.0)
    den_safe = jnp.where(den > 0.0, den, 1.0)
    losses = -jnp.log(num_safe / den_safe)
    out_ref[0, 0, 0] = jnp.sum(jnp.where(valid, losses, 0.0))
    out_ref[0, 0, 1] = jnp.sum(valid.astype(jnp.float32))


def _final_kernel(p_ref, out_ref):
    nb = p_ref.shape[0]
    loss = p_ref[0, 0, 0]
    cnt = p_ref[0, 0, 1]
    for k in range(1, nb):
        loss += p_ref[k, 0, 0]
        cnt += p_ref[k, 0, 1]
    out_ref[0, 0] = loss / jnp.maximum(cnt, 1.0)


def kernel(representations, ne_tags):
    n, d = representations.shape
    tags = ne_tags.astype(jnp.int32).reshape(1, n)

    xn = pl.pallas_call(
        _normalize_kernel,
        grid=(n // BNORM,),
        in_specs=[pl.BlockSpec((BNORM, d), lambda i: (i, 0))],
        out_specs=pl.BlockSpec((BNORM, d), lambda i: (i, 0)),
        out_shape=jax.ShapeDtypeStruct((n, d), jnp.bfloat16),
        compiler_params=pltpu.CompilerParams(
            dimension_semantics=("parallel",)),
    )(representations)

    ni = n // BM
    partials = pl.pallas_call(
        _row_block_kernel,
        grid=(ni,),
        in_specs=[
            pl.BlockSpec((BM, d), lambda i: (i, 0)),
            pl.BlockSpec((n, d), lambda i: (0, 0)),
            pl.BlockSpec((1, n), lambda i: (0, 0)),
        ],
        out_specs=pl.BlockSpec(
            (1, 1, 2), lambda i: (i, 0, 0), memory_space=pltpu.SMEM),
        out_shape=jax.ShapeDtypeStruct((ni, 1, 2), jnp.float32),
        compiler_params=pltpu.CompilerParams(
            dimension_semantics=("parallel",)),
    )(xn, xn, tags)

    out = pl.pallas_call(
        _final_kernel,
        in_specs=[pl.BlockSpec(memory_space=pltpu.SMEM)],
        out_specs=pl.BlockSpec(memory_space=pltpu.SMEM),
        out_shape=jax.ShapeDtypeStruct((1, 1), jnp.float32),
    )(partials)
    return out[0, 0]


# speedup vs baseline: 1.2036x; 1.0111x over previous
import jax
import jax.numpy as jnp
from jax.experimental import pallas as pl
from jax.experimental.pallas import tpu as pltpu

EPS = 1e-8
NTAGS = 8
SQRT_TINV = 3.798282186859221

BM = 1024
BNORM = 1024


def _normalize_kernel(x_ref, out_ref):
    x = x_ref[...]
    norm = jnp.sqrt(jnp.sum(x * x, axis=1, keepdims=True))
    scale = SQRT_TINV / jnp.maximum(norm, EPS)
    out_ref[...] = (x * scale).astype(jnp.bfloat16)


def _row_block_kernel(xn_ref, tags_ref, out_ref):
    i = pl.program_id(0)
    n = xn_ref.shape[0]

    xi = xn_ref[pl.ds(i * BM, BM), :]
    sim = jax.lax.dot_general(
        xi, xn_ref[...], (((1,), (1,)), ((), ())),
        preferred_element_type=jnp.float32)
    e_bf = jnp.exp2(sim).astype(jnp.bfloat16)

    ct = tags_ref[0, :]
    onehot = (ct[:, None] ==
              jax.lax.broadcasted_iota(jnp.int32, (n, NTAGS), 1)
              ).astype(jnp.bfloat16)
    r = jax.lax.dot_general(
        e_bf, onehot, (((1,), (0,)), ((), ())),
        preferred_element_type=jnp.float32)

    row_g = jax.lax.broadcasted_iota(jnp.int32, (BM, n), 0) + i * BM
    col_g = jax.lax.broadcasted_iota(jnp.int32, (BM, n), 1)
    de = jnp.sum(jnp.where(row_g == col_g, e_bf.astype(jnp.float32), 0.0),
                 axis=1, keepdims=True)

    rt = tags_ref[0, pl.ds(i * BM, BM)]
    sel = (rt[:, None] ==
           jax.lax.broadcasted_iota(jnp.int32, (BM, NTAGS), 1))
    den = jnp.sum(r, axis=1, keepdims=True) - de
    num = jnp.sum(jnp.where(sel, r, 0.0), axis=1, keepdims=True) - de
    valid = num > 0.0
    num_safe = jnp.where(valid, num, 1.0)
    den_safe = jnp.where(den > 0.0, den, 1.0)
    losses = -jnp.log(num_safe / den_safe)
    out_ref[0, 0, 0] = jnp.sum(jnp.where(valid, losses, 0.0))
    out_ref[0, 0, 1] = jnp.sum(valid.astype(jnp.float32))


def _final_kernel(p_ref, out_ref):
    nb = p_ref.shape[0]
    loss = p_ref[0, 0, 0]
    cnt = p_ref[0, 0, 1]
    for k in range(1, nb):
        loss += p_ref[k, 0, 0]
        cnt += p_ref[k, 0, 1]
    out_ref[0, 0] = loss / jnp.maximum(cnt, 1.0)


def kernel(representations, ne_tags):
    n, d = representations.shape
    tags = ne_tags.astype(jnp.int32).reshape(1, n)

    xn = pl.pallas_call(
        _normalize_kernel,
        grid=(n // BNORM,),
        in_specs=[pl.BlockSpec((BNORM, d), lambda i: (i, 0))],
        out_specs=pl.BlockSpec((BNORM, d), lambda i: (i, 0)),
        out_shape=jax.ShapeDtypeStruct((n, d), jnp.bfloat16),
        compiler_params=pltpu.CompilerParams(
            dimension_semantics=("parallel",)),
    )(representations)

    ni = n // BM
    partials = pl.pallas_call(
        _row_block_kernel,
        grid=(ni,),
        in_specs=[
            pl.BlockSpec((n, d), lambda i: (0, 0)),
            pl.BlockSpec((1, n), lambda i: (0, 0)),
        ],
        out_specs=pl.BlockSpec(
            (1, 1, 2), lambda i: (i, 0, 0), memory_space=pltpu.SMEM),
        out_shape=jax.ShapeDtypeStruct((ni, 1, 2), jnp.float32),
        compiler_params=pltpu.CompilerParams(
            dimension_semantics=("parallel",)),
    )(xn, tags)

    out = pl.pallas_call(
        _final_kernel,
        in_specs=[pl.BlockSpec(memory_space=pltpu.SMEM)],
        out_specs=pl.BlockSpec(memory_space=pltpu.SMEM),
        out_shape=jax.ShapeDtypeStruct((1, 1), jnp.float32),
    )(partials)
    return out[0, 0]


# speedup vs baseline: 1.3113x; 1.0894x over previous
import jax
import jax.numpy as jnp
from jax.experimental import pallas as pl
from jax.experimental.pallas import tpu as pltpu

EPS = 1e-8
NTAGS = 8
SQRT_TINV = 3.798282186859221

BM = 1024


def _row_block_kernel(x_ref, tags_ref, out_ref, xn_s):
    i = pl.program_id(0)
    n = x_ref.shape[0]

    @pl.when(i == 0)
    def _normalize():
        x = x_ref[...]
        norm = jnp.sqrt(jnp.sum(x * x, axis=1, keepdims=True))
        scale = SQRT_TINV / jnp.maximum(norm, EPS)
        xn_s[...] = (x * scale).astype(jnp.bfloat16)

    xi = xn_s[pl.ds(i * BM, BM), :]
    sim = jax.lax.dot_general(
        xi, xn_s[...], (((1,), (1,)), ((), ())),
        preferred_element_type=jnp.float32)
    e_bf = jnp.exp2(sim).astype(jnp.bfloat16)

    ct = tags_ref[0, :]
    onehot = (ct[:, None] ==
              jax.lax.broadcasted_iota(jnp.int32, (n, NTAGS), 1)
              ).astype(jnp.bfloat16)
    r = jax.lax.dot_general(
        e_bf, onehot, (((1,), (0,)), ((), ())),
        preferred_element_type=jnp.float32)

    row_g = jax.lax.broadcasted_iota(jnp.int32, (BM, n), 0) + i * BM
    col_g = jax.lax.broadcasted_iota(jnp.int32, (BM, n), 1)
    de = jnp.sum(jnp.where(row_g == col_g, e_bf.astype(jnp.float32), 0.0),
                 axis=1, keepdims=True)

    rt = tags_ref[0, pl.ds(i * BM, BM)]
    sel = (rt[:, None] ==
           jax.lax.broadcasted_iota(jnp.int32, (BM, NTAGS), 1))
    den = jnp.sum(r, axis=1, keepdims=True) - de
    num = jnp.sum(jnp.where(sel, r, 0.0), axis=1, keepdims=True) - de
    valid = num > 0.0
    num_safe = jnp.where(valid, num, 1.0)
    den_safe = jnp.where(den > 0.0, den, 1.0)
    losses = -jnp.log(num_safe / den_safe)
    out_ref[0, 0, 0] = jnp.sum(jnp.where(valid, losses, 0.0))
    out_ref[0, 0, 1] = jnp.sum(valid.astype(jnp.float32))


def _final_kernel(p_ref, out_ref):
    nb = p_ref.shape[0]
    loss = p_ref[0, 0, 0]
    cnt = p_ref[0, 0, 1]
    for k in range(1, nb):
        loss += p_ref[k, 0, 0]
        cnt += p_ref[k, 0, 1]
    out_ref[0, 0] = loss / jnp.maximum(cnt, 1.0)


def kernel(representations, ne_tags):
    n, d = representations.shape
    tags = ne_tags.astype(jnp.int32).reshape(1, n)

    ni = n // BM
    partials = pl.pallas_call(
        _row_block_kernel,
        grid=(ni,),
        in_specs=[
            pl.BlockSpec((n, d), lambda i: (0, 0)),
            pl.BlockSpec((1, n), lambda i: (0, 0)),
        ],
        out_specs=pl.BlockSpec(
            (1, 1, 2), lambda i: (i, 0, 0), memory_space=pltpu.SMEM),
        out_shape=jax.ShapeDtypeStruct((ni, 1, 2), jnp.float32),
        scratch_shapes=[
            pltpu.VMEM((n, d), jnp.bfloat16),
        ],
        compiler_params=pltpu.CompilerParams(
            dimension_semantics=("arbitrary",)),
    )(representations, tags)

    out = pl.pallas_call(
        _final_kernel,
        in_specs=[pl.BlockSpec(memory_space=pltpu.SMEM)],
        out_specs=pl.BlockSpec(memory_space=pltpu.SMEM),
        out_shape=jax.ShapeDtypeStruct((1, 1), jnp.float32),
    )(partials)
    return out[0, 0]


# speedup vs baseline: 1.3560x; 1.0341x over previous
import jax
import jax.numpy as jnp
from jax.experimental import pallas as pl
from jax.experimental.pallas import tpu as pltpu

EPS = 1e-8
NTAGS = 8
SQRT_TINV = 3.798282186859221
NEG_BIG = -1e30

BM = 1024


def _row_block_kernel(ni, x_ref, tags_ref, out_ref, xn_s):
    s = pl.program_id(0)
    n = xn_s.shape[0]

    @pl.when(s < ni)
    def _normalize_chunk():
        x = x_ref[...]
        norm = jnp.sqrt(jnp.sum(x * x, axis=1, keepdims=True))
        scale = SQRT_TINV / jnp.maximum(norm, EPS)
        xn_s[pl.ds(s * BM, BM), :] = (x * scale).astype(jnp.bfloat16)

    @pl.when(s >= ni)
    def _compute_block():
        i = s - ni
        xi = xn_s[pl.ds(i * BM, BM), :]
        sim = jax.lax.dot_general(
            xi, xn_s[...], (((1,), (1,)), ((), ())),
            preferred_element_type=jnp.float32)
        row_g = jax.lax.broadcasted_iota(jnp.int32, (BM, n), 0) + i * BM
        col_g = jax.lax.broadcasted_iota(jnp.int32, (BM, n), 1)
        e_bf = jnp.exp2(jnp.where(row_g == col_g, NEG_BIG, sim)
                        ).astype(jnp.bfloat16)

        ct = tags_ref[0, :]
        onehot = (ct[:, None] ==
                  jax.lax.broadcasted_iota(jnp.int32, (n, NTAGS), 1)
                  ).astype(jnp.bfloat16)
        r = jax.lax.dot_general(
            e_bf, onehot, (((1,), (0,)), ((), ())),
            preferred_element_type=jnp.float32)

        rt = tags_ref[0, pl.ds(i * BM, BM)]
        sel = (rt[:, None] ==
               jax.lax.broadcasted_iota(jnp.int32, (BM, NTAGS), 1))
        den = jnp.sum(r, axis=1, keepdims=True)
        num = jnp.sum(jnp.where(sel, r, 0.0), axis=1, keepdims=True)
        valid = num > 0.0
        num_safe = jnp.where(valid, num, 1.0)
        den_safe = jnp.where(den > 0.0, den, 1.0)
        losses = -jnp.log(num_safe / den_safe)
        out_ref[0, 0, 0] = jnp.sum(jnp.where(valid, losses, 0.0))
        out_ref[0, 0, 1] = jnp.sum(valid.astype(jnp.float32))


def _final_kernel(p_ref, out_ref):
    nb = p_ref.shape[0]
    loss = p_ref[0, 0, 0]
    cnt = p_ref[0, 0, 1]
    for k in range(1, nb):
        loss += p_ref[k, 0, 0]
        cnt += p_ref[k, 0, 1]
    out_ref[0, 0] = loss / jnp.maximum(cnt, 1.0)


def kernel(representations, ne_tags):
    n, d = representations.shape
    tags = ne_tags.astype(jnp.int32).reshape(1, n)
    ni = n // BM

    def body(*refs):
        _row_block_kernel(ni, *refs)

    partials = pl.pallas_call(
        body,
        grid=(2 * ni,),
        in_specs=[
            pl.BlockSpec((BM, d), lambda s: (jnp.minimum(s, ni - 1), 0)),
            pl.BlockSpec((1, n), lambda s: (0, 0)),
        ],
        out_specs=pl.BlockSpec(
            (1, 1, 2), lambda s: (jnp.maximum(s - ni, 0), 0, 0),
            memory_space=pltpu.SMEM),
        out_shape=jax.ShapeDtypeStruct((ni, 1, 2), jnp.float32),
        scratch_shapes=[
            pltpu.VMEM((n, d), jnp.bfloat16),
        ],
        compiler_params=pltpu.CompilerParams(
            dimension_semantics=("arbitrary",)),
    )(representations, tags)

    out = pl.pallas_call(
        _final_kernel,
        in_specs=[pl.BlockSpec(memory_space=pltpu.SMEM)],
        out_specs=pl.BlockSpec(memory_space=pltpu.SMEM),
        out_shape=jax.ShapeDtypeStruct((1, 1), jnp.float32),
    )(partials)
    return out[0, 0]
